# baseline (speedup 1.0000x reference)
import functools

import jax
import jax.numpy as jnp
from jax.experimental import pallas as pl
from jax.experimental.pallas import tpu as pltpu

_MARGIN = 0.2
_EPS = 1e-12


def _normalize(x, eps):
    n = jnp.sqrt(jnp.sum(x * x, axis=1, keepdims=True))
    return x / jnp.maximum(n, eps)


def _max_tree(parts):
    parts = list(parts)
    while len(parts) > 1:
        nxt = [jnp.maximum(parts[t], parts[t + 1])
               for t in range(0, len(parts) - 1, 2)]
        if len(parts) % 2:
            nxt.append(parts[-1])
        parts = nxt
    return parts[0]


def _fold_max(v, cj):
    return _max_tree([v[:, k * 128:(k + 1) * 128] for k in range(cj // 128)])


def _triplet_body(a_ref, pt_ref, pd_ref, out_ref, pnt_ref, mask_ref,
                  *, bm, cj, n, nj, d, margin, eps):
    i = pl.program_id(0)

    @pl.when(i == 0)
    def _setup():
        pt = pt_ref[...]
        nrm = jnp.sqrt(jnp.sum(pt * pt, axis=0, keepdims=True))
        pnt_ref[...] = pt / jnp.maximum(nrm, eps)
        row = jax.lax.broadcasted_iota(jnp.int32, (bm, 2 * cj), 0)
        col = jax.lax.broadcasted_iota(jnp.int32, (bm, 2 * cj), 1)
        mask_ref[...] = jnp.where(col == row + cj, -jnp.inf, 0.0)
        out_ref[...] = jnp.zeros_like(out_ref)

    a_n = _normalize(a_ref[...], eps)
    a_sq = jnp.sum(a_n * a_n, axis=1, keepdims=True)
    pdn = _normalize(pd_ref[...], eps)
    ap = jnp.sum((a_n - pdn) * (a_n - pdn), axis=1, keepdims=True)

    jd = (i * bm) // cj
    off = i * bm - jd * cj

    def tile(u):
        blk = jax.lax.rem(jd + u, nj)
        mm = jax.lax.dot_general(
            a_n, pnt_ref[:, pl.ds(blk * cj, cj)],
            (((1,), (0,)), ((), ())),
            preferred_element_type=jnp.float32)
        if u == 0:
            mm = mm + mask_ref[pl.ds(0, bm), pl.ds(cj - off, cj)]
        return _fold_max(mm, cj)

    folded = _max_tree([tile(u) for u in range(nj)])
    rowmax = jnp.max(folded, axis=1, keepdims=True)
    an_dist = a_sq + 1.0 - 2.0 * rowmax
    losses = jnp.maximum(ap - an_dist + margin, 0.0)
    out_ref[...] += jnp.sum(losses, keepdims=True) * (1.0 / n)


@jax.jit
def kernel(anchors, positives):
    n, d = anchors.shape
    bm = 512
    cj = 4096
    ni, nj = n // bm, n // cj
    body = functools.partial(_triplet_body, bm=bm, cj=cj, n=n, nj=nj, d=d,
                             margin=_MARGIN, eps=_EPS)
    out = pl.pallas_call(
        body,
        grid=(ni,),
        in_specs=[
            pl.BlockSpec((bm, d), lambda i: (i, 0)),
            pl.BlockSpec((d, n), lambda i: (0, 0)),
            pl.BlockSpec((bm, d), lambda i: (i, 0)),
        ],
        out_specs=pl.BlockSpec((1, 1), lambda i: (0, 0)),
        out_shape=jax.ShapeDtypeStruct((1, 1), jnp.float32),
        scratch_shapes=[
            pltpu.VMEM((d, n), jnp.float32),
            pltpu.VMEM((bm, 2 * cj), jnp.float32),
        ],
    )(anchors, positives.T, positives)
    return out[0, 0]


# speedup vs baseline: 1.0046x; 1.0046x over previous
import functools

import jax
import jax.numpy as jnp
from jax.experimental import pallas as pl
from jax.experimental.pallas import tpu as pltpu

_MARGIN = 0.2
_EPS = 1e-12


def _normalize(x, eps):
    n = jnp.sqrt(jnp.sum(x * x, axis=1, keepdims=True))
    return x / jnp.maximum(n, eps)


def _max_tree(parts):
    parts = list(parts)
    while len(parts) > 1:
        nxt = [jnp.maximum(parts[t], parts[t + 1])
               for t in range(0, len(parts) - 1, 2)]
        if len(parts) % 2:
            nxt.append(parts[-1])
        parts = nxt
    return parts[0]


def _fold_max(v, cj):
    return _max_tree([v[:, k * 128:(k + 1) * 128] for k in range(cj // 128)])


def _triplet_body(a_ref, pt_ref, pd_ref, out_ref, pnt_ref, mask_ref,
                  *, bm, cj, n, nj, d, margin, eps):
    i = pl.program_id(0)

    @pl.when(i == 0)
    def _setup():
        pt = pt_ref[...]
        nrm = jnp.sqrt(jnp.sum(pt * pt, axis=0, keepdims=True))
        pnt_ref[...] = pt / jnp.maximum(nrm, eps)
        row = jax.lax.broadcasted_iota(jnp.int32, (bm, 2 * cj), 0)
        col = jax.lax.broadcasted_iota(jnp.int32, (bm, 2 * cj), 1)
        mask_ref[...] = jnp.where(col == row + cj, -jnp.inf, 0.0)
        out_ref[...] = jnp.zeros_like(out_ref)

    a_n = _normalize(a_ref[...], eps)
    a_sq = jnp.sum(a_n * a_n, axis=1, keepdims=True)
    pdn = _normalize(pd_ref[...], eps)
    ap = jnp.sum((a_n - pdn) * (a_n - pdn), axis=1, keepdims=True)

    jd = (i * bm) // cj
    off = i * bm - jd * cj

    def tile(u):
        blk = jax.lax.rem(jd + u, nj)
        mm = jax.lax.dot_general(
            a_n, pnt_ref[:, pl.ds(blk * cj, cj)],
            (((1,), (0,)), ((), ())),
            preferred_element_type=jnp.float32)
        if u == 0:
            mm = mm + mask_ref[pl.ds(0, bm), pl.ds(cj - off, cj)]
        return _fold_max(mm, cj)

    folded = _max_tree([tile(u) for u in range(nj)])
    rowmax = jnp.max(folded, axis=1, keepdims=True)
    an_dist = a_sq + 1.0 - 2.0 * rowmax
    losses = jnp.maximum(ap - an_dist + margin, 0.0)
    out_ref[...] += jnp.sum(losses, keepdims=True) * (1.0 / n)


@jax.jit
def kernel(anchors, positives):
    n, d = anchors.shape
    bm = 512
    cj = 1024
    ni, nj = n // bm, n // cj
    body = functools.partial(_triplet_body, bm=bm, cj=cj, n=n, nj=nj, d=d,
                             margin=_MARGIN, eps=_EPS)
    out = pl.pallas_call(
        body,
        grid=(ni,),
        in_specs=[
            pl.BlockSpec((bm, d), lambda i: (i, 0)),
            pl.BlockSpec((d, n), lambda i: (0, 0)),
            pl.BlockSpec((bm, d), lambda i: (i, 0)),
        ],
        out_specs=pl.BlockSpec((1, 1), lambda i: (0, 0)),
        out_shape=jax.ShapeDtypeStruct((1, 1), jnp.float32),
        scratch_shapes=[
            pltpu.VMEM((d, n), jnp.float32),
            pltpu.VMEM((bm, 2 * cj), jnp.float32),
        ],
    )(anchors, positives.T, positives)
    return out[0, 0]


# speedup vs baseline: 1.0061x; 1.0015x over previous
import functools

import jax
import jax.numpy as jnp
from jax.experimental import pallas as pl
from jax.experimental.pallas import tpu as pltpu

_MARGIN = 0.2
_EPS = 1e-12


def _normalize(x, eps):
    n = jnp.sqrt(jnp.sum(x * x, axis=1, keepdims=True))
    return x / jnp.maximum(n, eps)


def _max_tree(parts):
    parts = list(parts)
    while len(parts) > 1:
        nxt = [jnp.maximum(parts[t], parts[t + 1])
               for t in range(0, len(parts) - 1, 2)]
        if len(parts) % 2:
            nxt.append(parts[-1])
        parts = nxt
    return parts[0]


def _fold_max(v, cj):
    return _max_tree([v[:, k * 128:(k + 1) * 128] for k in range(cj // 128)])


def _triplet_body(a_ref, pt_ref, pd_ref, out_ref, pnt_ref, mask_ref,
                  *, bm, cj, n, nj, d, margin, eps):
    i = pl.program_id(0)

    @pl.when(i == 0)
    def _setup():
        pt = pt_ref[...]
        nrm = jnp.sqrt(jnp.sum(pt * pt, axis=0, keepdims=True))
        pnt_ref[...] = pt / jnp.maximum(nrm, eps)
        row = jax.lax.broadcasted_iota(jnp.int32, (bm, 2 * cj), 0)
        col = jax.lax.broadcasted_iota(jnp.int32, (bm, 2 * cj), 1)
        mask_ref[...] = jnp.where(col == row + cj, -jnp.inf, 0.0)
        out_ref[...] = jnp.zeros_like(out_ref)

    a_n = _normalize(a_ref[...], eps)
    a_sq = jnp.sum(a_n * a_n, axis=1, keepdims=True)
    pdn = _normalize(pd_ref[...], eps)
    ap = jnp.sum((a_n - pdn) * (a_n - pdn), axis=1, keepdims=True)

    jd = (i * bm) // cj
    off = i * bm - jd * cj

    def tile(u):
        blk = jax.lax.rem(jd + u, nj)
        mm = jax.lax.dot_general(
            a_n, pnt_ref[:, pl.ds(blk * cj, cj)],
            (((1,), (0,)), ((), ())),
            preferred_element_type=jnp.float32)
        if u == 0:
            mm = mm + mask_ref[pl.ds(0, bm), pl.ds(cj - off, cj)]
        return _fold_max(mm, cj)

    folded = _max_tree([tile(u) for u in range(nj)])
    rowmax = jnp.max(folded, axis=1, keepdims=True)
    an_dist = a_sq + 1.0 - 2.0 * rowmax
    losses = jnp.maximum(ap - an_dist + margin, 0.0)
    out_ref[...] += jnp.sum(losses, keepdims=True) * (1.0 / n)


@jax.jit
def kernel(anchors, positives):
    n, d = anchors.shape
    bm = 512
    cj = 512
    ni, nj = n // bm, n // cj
    body = functools.partial(_triplet_body, bm=bm, cj=cj, n=n, nj=nj, d=d,
                             margin=_MARGIN, eps=_EPS)
    out = pl.pallas_call(
        body,
        grid=(ni,),
        in_specs=[
            pl.BlockSpec((bm, d), lambda i: (i, 0)),
            pl.BlockSpec((d, n), lambda i: (0, 0)),
            pl.BlockSpec((bm, d), lambda i: (i, 0)),
        ],
        out_specs=pl.BlockSpec((1, 1), lambda i: (0, 0)),
        out_shape=jax.ShapeDtypeStruct((1, 1), jnp.float32),
        scratch_shapes=[
            pltpu.VMEM((d, n), jnp.float32),
            pltpu.VMEM((bm, 2 * cj), jnp.float32),
        ],
    )(anchors, positives.T, positives)
    return out[0, 0]


# speedup vs baseline: 1.0484x; 1.0420x over previous
import functools

import jax
import jax.numpy as jnp
from jax.experimental import pallas as pl
from jax.experimental.pallas import tpu as pltpu

_MARGIN = 0.2
_EPS = 1e-12


def _normalize(x, eps):
    n = jnp.sqrt(jnp.sum(x * x, axis=1, keepdims=True))
    return x / jnp.maximum(n, eps)


def _max_tree(parts):
    parts = list(parts)
    while len(parts) > 1:
        nxt = [jnp.maximum(parts[t], parts[t + 1])
               for t in range(0, len(parts) - 1, 2)]
        if len(parts) % 2:
            nxt.append(parts[-1])
        parts = nxt
    return parts[0]


def _fold_max(v, cj):
    return _max_tree([v[:, k * 128:(k + 1) * 128] for k in range(cj // 128)])


def _triplet_body(a_ref, pt_ref, pd_ref, out_ref, pnt_ref, mask_ref,
                  *, bm, cj, n, nj, d, margin, eps):
    i = pl.program_id(0)

    @pl.when(i == 0)
    def _setup():
        pt = pt_ref[...]
        nrm = jnp.sqrt(jnp.sum(pt * pt, axis=0, keepdims=True))
        pnt_ref[...] = pt / jnp.maximum(nrm, eps)
        row = jax.lax.broadcasted_iota(jnp.int32, (bm, 2 * cj), 0)
        col = jax.lax.broadcasted_iota(jnp.int32, (bm, 2 * cj), 1)
        mask_ref[...] = jnp.where(col == row + cj, -jnp.inf, 0.0)
        out_ref[...] = jnp.zeros_like(out_ref)

    def row_block(a_blk, pd_blk, gi):
        a_n = _normalize(a_blk, eps)
        a_sq = jnp.sum(a_n * a_n, axis=1, keepdims=True)
        pdn = _normalize(pd_blk, eps)
        ap = jnp.sum((a_n - pdn) * (a_n - pdn), axis=1, keepdims=True)

        jd = (gi * bm) // cj
        off = gi * bm - jd * cj

        def tile(u):
            blk = jax.lax.rem(jd + u, nj)
            mm = jax.lax.dot_general(
                a_n, pnt_ref[:, pl.ds(blk * cj, cj)],
                (((1,), (0,)), ((), ())),
                preferred_element_type=jnp.float32)
            if u == 0:
                mm = mm + mask_ref[pl.ds(0, bm), pl.ds(cj - off, cj)]
            return _fold_max(mm, cj)

        folded = _max_tree([tile(u) for u in range(nj)])
        rowmax = jnp.max(folded, axis=1, keepdims=True)
        an_dist = a_sq + 1.0 - 2.0 * rowmax
        losses = jnp.maximum(ap - an_dist + margin, 0.0)
        return jnp.sum(losses, keepdims=True)

    part0 = row_block(a_ref[:bm, :], pd_ref[:bm, :], 2 * i)
    part1 = row_block(a_ref[bm:, :], pd_ref[bm:, :], 2 * i + 1)
    out_ref[...] += (part0 + part1) * (1.0 / n)


@jax.jit
def kernel(anchors, positives):
    n, d = anchors.shape
    bm = 512
    cj = 1024
    ni, nj = n // (2 * bm), n // cj
    body = functools.partial(_triplet_body, bm=bm, cj=cj, n=n, nj=nj, d=d,
                             margin=_MARGIN, eps=_EPS)
    out = pl.pallas_call(
        body,
        grid=(ni,),
        in_specs=[
            pl.BlockSpec((2 * bm, d), lambda i: (i, 0)),
            pl.BlockSpec((d, n), lambda i: (0, 0)),
            pl.BlockSpec((2 * bm, d), lambda i: (i, 0)),
        ],
        out_specs=pl.BlockSpec((1, 1), lambda i: (0, 0)),
        out_shape=jax.ShapeDtypeStruct((1, 1), jnp.float32),
        scratch_shapes=[
            pltpu.VMEM((d, n), jnp.float32),
            pltpu.VMEM((bm, 2 * cj), jnp.float32),
        ],
    )(anchors, positives.T, positives)
    return out[0, 0]


# speedup vs baseline: 1.0518x; 1.0032x over previous
import functools

import jax
import jax.numpy as jnp
from jax.experimental import pallas as pl
from jax.experimental.pallas import tpu as pltpu

_MARGIN = 0.2
_EPS = 1e-12


def _normalize(x, eps):
    n = jnp.sqrt(jnp.sum(x * x, axis=1, keepdims=True))
    return x / jnp.maximum(n, eps)


def _max_tree(parts):
    parts = list(parts)
    while len(parts) > 1:
        nxt = [jnp.maximum(parts[t], parts[t + 1])
               for t in range(0, len(parts) - 1, 2)]
        if len(parts) % 2:
            nxt.append(parts[-1])
        parts = nxt
    return parts[0]


def _fold_max(v, cj):
    return _max_tree([v[:, k * 128:(k + 1) * 128] for k in range(cj // 128)])


def _triplet_body(a_ref, pt_ref, pd_ref, out_ref, pnt_ref, mask_ref,
                  *, bm, cj, n, nj, d, margin, eps):
    i = pl.program_id(0)

    @pl.when(i == 0)
    def _setup():
        pt = pt_ref[...]
        nrm = jnp.sqrt(jnp.sum(pt * pt, axis=0, keepdims=True))
        pnt_ref[...] = pt / jnp.maximum(nrm, eps)
        row = jax.lax.broadcasted_iota(jnp.int32, (bm, 2 * cj), 0)
        col = jax.lax.broadcasted_iota(jnp.int32, (bm, 2 * cj), 1)
        mask_ref[...] = jnp.where(col == row + cj, -jnp.inf, 0.0)
        out_ref[...] = jnp.zeros_like(out_ref)

    def row_block(a_blk, pd_blk, gi):
        a_n = _normalize(a_blk, eps)
        a_sq = jnp.sum(a_n * a_n, axis=1, keepdims=True)
        pdn = _normalize(pd_blk, eps)
        ap = jnp.sum((a_n - pdn) * (a_n - pdn), axis=1, keepdims=True)

        jd = (gi * bm) // cj
        off = gi * bm - jd * cj

        def tile(u):
            blk = jax.lax.rem(jd + u, nj)
            mm = jax.lax.dot_general(
                a_n, pnt_ref[:, pl.ds(blk * cj, cj)],
                (((1,), (0,)), ((), ())),
                preferred_element_type=jnp.float32)
            if u == 0:
                mm = mm + mask_ref[pl.ds(0, bm), pl.ds(cj - off, cj)]
            return _fold_max(mm, cj)

        folded = _max_tree([tile(u) for u in range(nj)])
        rowmax = jnp.max(folded, axis=1, keepdims=True)
        an_dist = a_sq + 1.0 - 2.0 * rowmax
        losses = jnp.maximum(ap - an_dist + margin, 0.0)
        return jnp.sum(losses, keepdims=True)

    part0 = row_block(a_ref[:bm, :], pd_ref[:bm, :], 2 * i)
    part1 = row_block(a_ref[bm:, :], pd_ref[bm:, :], 2 * i + 1)
    out_ref[...] += (part0 + part1) * (1.0 / n)


@jax.jit
def kernel(anchors, positives):
    n, d = anchors.shape
    bm = 512
    cj = 512
    ni, nj = n // (2 * bm), n // cj
    body = functools.partial(_triplet_body, bm=bm, cj=cj, n=n, nj=nj, d=d,
                             margin=_MARGIN, eps=_EPS)
    out = pl.pallas_call(
        body,
        grid=(ni,),
        in_specs=[
            pl.BlockSpec((2 * bm, d), lambda i: (i, 0)),
            pl.BlockSpec((d, n), lambda i: (0, 0)),
            pl.BlockSpec((2 * bm, d), lambda i: (i, 0)),
        ],
        out_specs=pl.BlockSpec((1, 1), lambda i: (0, 0)),
        out_shape=jax.ShapeDtypeStruct((1, 1), jnp.float32),
        scratch_shapes=[
            pltpu.VMEM((d, n), jnp.float32),
            pltpu.VMEM((bm, 2 * cj), jnp.float32),
        ],
    )(anchors, positives.T, positives)
    return out[0, 0]
